# baseline (speedup 1.0000x reference)
import math

import jax
import jax.numpy as jnp
from jax.experimental import pallas as pl

B, S, NE, N1, N2, KV, QD = 4, 512, 512, 8, 8, 100, 768
EB = 64
CW = 256


def _att_body(q0_ref, k1_ref, v1_ref, k2_ref, v2_ref,
              wq1_ref, wkv1_ref, bq1_ref, wq2_ref, wkv2_ref, bq2_ref,
              out_ref):
    f32 = jnp.float32
    q0 = q0_ref[0]

    def qproj(wq_ref, b_ref, wkv_ref):
        qh = jnp.tanh(
            jax.lax.dot_general(q0, wq_ref[...],
                                (((1,), (1,)), ((), ())),
                                preferred_element_type=f32)
            + b_ref[...])
        return jax.lax.dot_general(qh, wkv_ref[...],
                                   (((1,), (0,)), ((), ())),
                                   preferred_element_type=f32)

    qw1 = qproj(wq1_ref, bq1_ref, wkv1_ref)
    qw2 = qproj(wq2_ref, bq2_ref, wkv2_ref)

    def att_weights(scores):
        n = scores.shape[1]
        a = jnp.where(scores == 0.0, -10000.0, scores)
        a = jnp.where(a >= 0.0, a, 0.01 * a)
        m = jnp.max(a, axis=1, keepdims=True)
        e = jnp.exp(a - m)
        p = e / jnp.sum(e, axis=1, keepdims=True)
        return jnp.where(p == 1.0 / n, 0.0, p)

    k2 = k2_ref[0].reshape(EB * N1, N2, KV)
    s2 = jnp.sum(k2 * qw2.reshape(1, 1, KV), axis=2) / math.sqrt(KV)
    p2 = att_weights(s2)
    v2 = v2_ref[0].reshape(EB * N1, N2, KV)
    sent2 = jnp.sum(v2 * p2[:, :, None], axis=1)

    k1 = k1_ref[0].reshape(EB, N1, KV)
    s1 = jnp.sum(k1 * qw1.reshape(1, 1, KV), axis=2) / math.sqrt(KV)
    p1 = att_weights(s1)
    v1 = v1_ref[0].reshape(EB, N1, KV)
    c1 = jnp.sum(v1 * p1[:, :, None], axis=1)
    c2 = jnp.sum(sent2.reshape(EB, N1, KV) * p1[:, :, None], axis=1)
    pad = jnp.zeros((EB, CW - 2 * KV), f32)
    out_ref[0] = jnp.concatenate([c1, c2, pad], axis=1)


def _attention(q0, k1r, v1r, k2r, v2r, W_kv1, W_kv2, W_q1, b_q1, W_q2, b_q2,
               interpret=False):
    grid = (B, NE // EB)
    specs = [
        pl.BlockSpec((1, 1, QD), lambda b, e: (b, 0, 0)),
        pl.BlockSpec((1, EB * N1, KV), lambda b, e: (b, e, 0)),
        pl.BlockSpec((1, EB * N1, KV), lambda b, e: (b, e, 0)),
        pl.BlockSpec((1, EB * N1 * N2, KV), lambda b, e: (b, e, 0)),
        pl.BlockSpec((1, EB * N1 * N2, KV), lambda b, e: (b, e, 0)),
        pl.BlockSpec((KV, QD), lambda b, e: (0, 0)),
        pl.BlockSpec((KV, KV), lambda b, e: (0, 0)),
        pl.BlockSpec((1, KV), lambda b, e: (0, 0)),
        pl.BlockSpec((KV, QD), lambda b, e: (0, 0)),
        pl.BlockSpec((KV, KV), lambda b, e: (0, 0)),
        pl.BlockSpec((1, KV), lambda b, e: (0, 0)),
    ]
    return pl.pallas_call(
        _att_body,
        grid=grid,
        in_specs=[specs[0], specs[1], specs[2], specs[3], specs[4],
                  specs[5], specs[6], specs[7], specs[8], specs[9], specs[10]],
        out_specs=pl.BlockSpec((1, EB, CW), lambda b, e: (b, e, 0)),
        out_shape=jax.ShapeDtypeStruct((B, NE, CW), jnp.float32),
        interpret=interpret,
    )(q0, k1r, v1r, k2r, v2r, W_q1, W_kv1, b_q1, W_q2, W_kv2, b_q2)


def kernel(input_ent, q, k_1, v_1, k_2, v_2,
           W_kv1, W_kv2, W_q1, b_q1, W_q2, b_q2, interpret=False):
    q0 = q[:, 0, :].reshape(B, 1, QD)
    k1r = k_1.reshape(B, NE * N1, KV)
    v1r = v_1.reshape(B, NE * N1, KV)
    k2r = k_2.reshape(B, NE * N1 * N2, KV)
    v2r = v_2.reshape(B, NE * N1 * N2, KV)
    combined = _attention(q0, k1r, v1r, k2r, v2r,
                          W_kv1, W_kv2, W_q1, b_q1.reshape(1, KV),
                          W_q2, b_q2.reshape(1, KV),
                          interpret=interpret)

    mask = input_ent != 0
    rank = jnp.cumsum(mask.astype(jnp.int32), axis=1) - 1
    gathered = jnp.take_along_axis(combined, jnp.clip(rank, 0)[:, :, None],
                                   axis=1)
    out = jnp.where(mask[:, :, None], gathered, 0.0)
    return out[:, :, : 2 * KV]
